# baseline (speedup 1.0000x reference)
import jax
import jax.numpy as jnp
from jax import lax
from jax.experimental import pallas as pl
from jax.experimental.pallas import tpu as pltpu
from jax.experimental.pallas import tpu_sc as plsc

N_NODES = 10000
N_EDGES = 320000
IN_FEATS = 128
HIDDEN = 16
OUT_FEATS = 2

NC, NS = 2, 16
NW = NC * NS
NP = 10240
NQ = NP // 8
RPT = NP // NS
EPW = N_EDGES // NW
K = 80
NCHUNK = EPW // K
SBC = 25
NSB = NCHUNK // SBC


def _sc_mesh():
    return plsc.VectorSubcoreMesh(
        core_axis_name="c", subcore_axis_name="s",
        num_cores=NC, num_subcores=NS)


def _deg_body(dst_hbm, ones_hbm, zeros_hbm, out_hbm, dst_v, ones_v, acc_sh,
              ssem):
    c = lax.axis_index("c")
    s = lax.axis_index("s")
    wid = s * NC + c
    pltpu.sync_copy(zeros_hbm, acc_sh.at[pl.ds(s * RPT, RPT)])
    pltpu.sync_copy(dst_hbm.at[pl.ds(wid * EPW, EPW)], dst_v)
    pltpu.sync_copy(ones_hbm, ones_v)
    plsc.subcore_barrier()

    def fire(j, carry):
        pltpu.async_copy(ones_v, acc_sh.at[dst_v.at[pl.ds(j * K, K)]],
                         ssem, add=True)
        return carry
    lax.fori_loop(0, NCHUNK, fire, 0)

    def drain(j, carry):
        pltpu.make_async_copy(ones_hbm, ones_v, ssem).wait()
        return carry
    lax.fori_loop(0, NCHUNK, drain, 0)

    plsc.subcore_barrier()
    pltpu.sync_copy(acc_sh.at[pl.ds(s * RPT, RPT)],
                    out_hbm.at[c].at[pl.ds(s * RPT, RPT)])


def _agg_body(rows_hbm, src_hbm, dst_hbm, zeros_hbm, out_hbm,
              src_v, dst_v, bufs, acc_sh, gsems, ssems):
    c = lax.axis_index("c")
    s = lax.axis_index("s")
    wid = s * NC + c
    pltpu.sync_copy(zeros_hbm, acc_sh.at[pl.ds(s * RPT, RPT)])
    pltpu.sync_copy(src_hbm.at[pl.ds(wid * EPW, EPW)], src_v)
    pltpu.sync_copy(dst_hbm.at[pl.ds(wid * EPW, EPW)], dst_v)
    plsc.subcore_barrier()

    def fire_gathers(sb, bi):
        base = sb * SBC
        def f(j, carry):
            idx = src_v.at[pl.ds((base + j) * K, K)]
            pltpu.async_copy(rows_hbm.at[idx], bufs.at[bi].at[j], gsems.at[bi])
            return carry
        lax.fori_loop(0, SBC, f, 0)

    def fire_scatters(sb, bi):
        base = sb * SBC
        def f(j, carry):
            idx = dst_v.at[pl.ds((base + j) * K, K)]
            pltpu.async_copy(bufs.at[bi].at[j], acc_sh.at[idx],
                             ssems.at[bi], add=True)
            return carry
        lax.fori_loop(0, SBC, f, 0)

    def drain(sem):
        def f(j, carry):
            pltpu.make_async_copy(rows_hbm.at[pl.ds(0, K)],
                                  bufs.at[0].at[0], sem).wait()
            return carry
        lax.fori_loop(0, SBC, f, 0)

    fire_gathers(0, 0)
    for sb in range(NSB):
        bi = sb % 2
        drain(gsems.at[bi])
        if sb >= 1:
            drain(ssems.at[(sb - 1) % 2])
        if sb + 1 < NSB:
            fire_gathers(sb + 1, (sb + 1) % 2)
        fire_scatters(sb, bi)
    drain(ssems.at[(NSB - 1) % 2])

    plsc.subcore_barrier()
    pltpu.sync_copy(acc_sh.at[pl.ds(s * RPT, RPT)],
                    out_hbm.at[c].at[pl.ds(s * RPT, RPT)])


_sc_params = pltpu.CompilerParams(use_tc_tiling_on_sc=False)

_deg_call = pl.kernel(
    _deg_body,
    out_type=jax.ShapeDtypeStruct((NC, NP, HIDDEN), jnp.float32),
    mesh=_sc_mesh(),
    compiler_params=_sc_params,
    scratch_types=[
        pltpu.VMEM((EPW,), jnp.int32),
        pltpu.VMEM((K, HIDDEN), jnp.float32),
        pltpu.VMEM_SHARED((NP, HIDDEN), jnp.float32),
        pltpu.SemaphoreType.DMA,
    ],
)

_agg_call = pl.kernel(
    _agg_body,
    out_type=jax.ShapeDtypeStruct((NC, NP, HIDDEN), jnp.float32),
    mesh=_sc_mesh(),
    compiler_params=_sc_params,
    scratch_types=[
        pltpu.VMEM((EPW,), jnp.int32),
        pltpu.VMEM((EPW,), jnp.int32),
        pltpu.VMEM((2, SBC, K, HIDDEN), jnp.float32),
        pltpu.VMEM_SHARED((NP, HIDDEN), jnp.float32),
        pltpu.SemaphoreType.DMA((2,)),
        pltpu.SemaphoreType.DMA((2,)),
    ],
)


def _tc1_body(x, w1, degq, hsq, dinvq):
    dinv = lax.rsqrt(degq[0] + degq[1] + 1.0)
    dinvq[...] = dinv
    h = jnp.dot(x[...], w1[...], preferred_element_type=jnp.float32)
    blocks = [h[i * NQ:(i + 1) * NQ, :] for i in range(7)]
    tail = jnp.concatenate(
        [h[7 * NQ:N_NODES, :],
         jnp.zeros((NP - N_NODES, HIDDEN), jnp.float32)], axis=0)
    hp = jnp.concatenate(blocks + [tail], axis=1)
    hsq[...] = hp * dinv


def _tc2_body(a1p, hsq, dinvq, b1t, gsq):
    a1 = (a1p[0] + a1p[1] + hsq[...]) * dinvq[...] + b1t[...]
    gsq[...] = jnp.maximum(a1, 0.0) * dinvq[...]


def _tc3_body(a2p, gsq, dinvq, w2b, b2t, out):
    a2 = (a2p[0] + a2p[1] + gsq[...]) * dinvq[...]
    out[...] = jnp.dot(a2, w2b[...], preferred_element_type=jnp.float32) \
        + b2t[...]


_tc1 = pl.pallas_call(
    _tc1_body,
    out_shape=[jax.ShapeDtypeStruct((NQ, 128), jnp.float32),
               jax.ShapeDtypeStruct((NQ, 128), jnp.float32)],
)

_tc2 = pl.pallas_call(
    _tc2_body,
    out_shape=jax.ShapeDtypeStruct((NQ, 128), jnp.float32),
)

_tc3 = pl.pallas_call(
    _tc3_body,
    out_shape=jax.ShapeDtypeStruct((NQ, 8 * OUT_FEATS), jnp.float32),
)


def kernel(x, edge_index, W1, b1, W2, b2):
    e32 = edge_index.astype(jnp.int32)
    t = e32 >> 8
    d = ((t << 3) + (t << 2) + t) >> 6
    m = e32 - ((d << 10) + (d << 8))
    ep = (m << 3) + d
    src_p, dst_p = ep[0], ep[1]
    w2b = jnp.kron(jnp.eye(8, dtype=jnp.float32), W2)
    b1t = jnp.tile(b1, 8).reshape(1, 128)
    b2t = jnp.tile(b2, 8).reshape(1, 8 * OUT_FEATS)
    onesKH = jnp.ones((K, HIDDEN), jnp.float32)
    zerosRH = jnp.zeros((RPT, HIDDEN), jnp.float32)

    degp = _deg_call(dst_p, onesKH, zerosRH)
    hsq, dinvq = _tc1(x, W1, degp.reshape(NC, NQ, 128))
    a1p = _agg_call(hsq.reshape(NP, HIDDEN), src_p, dst_p, zerosRH)
    gsq = _tc2(a1p.reshape(NC, NQ, 128), hsq, dinvq, b1t)
    a2p = _agg_call(gsq.reshape(NP, HIDDEN), src_p, dst_p, zerosRH)
    outq = _tc3(a2p.reshape(NC, NQ, 128), gsq, dinvq, w2b, b2t)
    o = outq.reshape(NQ, 8, OUT_FEATS).transpose(1, 0, 2)
    return o.reshape(NP, OUT_FEATS)[:N_NODES]


# speedup vs baseline: 1.0421x; 1.0421x over previous
import jax
import jax.numpy as jnp
from jax import lax
from jax.experimental import pallas as pl
from jax.experimental.pallas import tpu as pltpu
from jax.experimental.pallas import tpu_sc as plsc

N_NODES = 10000
N_EDGES = 320000
IN_FEATS = 128
HIDDEN = 16
OUT_FEATS = 2

NC, NS = 2, 16
NW = NC * NS
NP = 10240
NQ = NP // 8
RPT = NP // NS
EPW = N_EDGES // NW
K = 80
NCHUNK = EPW // K
SBC = 25
NSB = NCHUNK // SBC


def _sc_mesh():
    return plsc.VectorSubcoreMesh(
        core_axis_name="c", subcore_axis_name="s",
        num_cores=NC, num_subcores=NS)


def _deg_body(dst_hbm, ones_hbm, zeros_hbm, out_hbm, dst_v, ones_v, acc_sh,
              ssem):
    c = lax.axis_index("c")
    s = lax.axis_index("s")
    wid = s * NC + c
    pltpu.sync_copy(zeros_hbm, acc_sh.at[pl.ds(s * RPT, RPT)])
    pltpu.sync_copy(dst_hbm.at[pl.ds(wid * EPW, EPW)], dst_v)
    pltpu.sync_copy(ones_hbm, ones_v)
    plsc.subcore_barrier()

    def fire(j, carry):
        pltpu.async_copy(ones_v, acc_sh.at[dst_v.at[pl.ds(j * K, K)]],
                         ssem, add=True)
        return carry
    lax.fori_loop(0, NCHUNK, fire, 0)

    def drain(j, carry):
        pltpu.make_async_copy(ones_hbm, ones_v, ssem).wait()
        return carry
    lax.fori_loop(0, NCHUNK, drain, 0)

    plsc.subcore_barrier()
    pltpu.sync_copy(acc_sh.at[pl.ds(s * RPT, RPT)],
                    out_hbm.at[c].at[pl.ds(s * RPT, RPT)])


def _agg_body(rows_hbm, src_hbm, dst_hbm, zeros_hbm, out_hbm,
              src_v, dst_v, bufs, acc_sh, gsems, ssems):
    c = lax.axis_index("c")
    s = lax.axis_index("s")
    wid = s * NC + c
    pltpu.sync_copy(zeros_hbm, acc_sh.at[pl.ds(s * RPT, RPT)])
    pltpu.sync_copy(src_hbm.at[pl.ds(wid * EPW, EPW)], src_v)
    pltpu.sync_copy(dst_hbm.at[pl.ds(wid * EPW, EPW)], dst_v)
    plsc.subcore_barrier()

    def fire_gathers(sb, bi):
        base = sb * SBC
        def f(j, carry):
            idx = src_v.at[pl.ds((base + j) * K, K)]
            pltpu.async_copy(rows_hbm.at[idx], bufs.at[bi].at[j], gsems.at[bi])
            return carry
        lax.fori_loop(0, SBC, f, 0)

    def fire_scatters(sb, bi):
        base = sb * SBC
        def f(j, carry):
            idx = dst_v.at[pl.ds((base + j) * K, K)]
            pltpu.async_copy(bufs.at[bi].at[j], acc_sh.at[idx],
                             ssems.at[bi], add=True)
            return carry
        lax.fori_loop(0, SBC, f, 0)

    def drain(sem):
        def f(j, carry):
            pltpu.make_async_copy(rows_hbm.at[pl.ds(0, K)],
                                  bufs.at[0].at[0], sem).wait()
            return carry
        lax.fori_loop(0, SBC, f, 0)

    fire_gathers(0, 0)
    for sb in range(NSB):
        bi = sb % 2
        drain(gsems.at[bi])
        if sb >= 1:
            drain(ssems.at[(sb - 1) % 2])
        if sb + 1 < NSB:
            fire_gathers(sb + 1, (sb + 1) % 2)
        fire_scatters(sb, bi)
    drain(ssems.at[(NSB - 1) % 2])

    plsc.subcore_barrier()
    pltpu.sync_copy(acc_sh.at[pl.ds(s * RPT, RPT)],
                    out_hbm.at[c].at[pl.ds(s * RPT, RPT)])


_sc_params = pltpu.CompilerParams(use_tc_tiling_on_sc=False)

_deg_call = pl.kernel(
    _deg_body,
    out_type=jax.ShapeDtypeStruct((NC, NP, HIDDEN), jnp.float32),
    mesh=_sc_mesh(),
    compiler_params=_sc_params,
    scratch_types=[
        pltpu.VMEM((EPW,), jnp.int32),
        pltpu.VMEM((K, HIDDEN), jnp.float32),
        pltpu.VMEM_SHARED((NP, HIDDEN), jnp.float32),
        pltpu.SemaphoreType.DMA,
    ],
)

_agg_call = pl.kernel(
    _agg_body,
    out_type=jax.ShapeDtypeStruct((NC, NP, HIDDEN), jnp.float32),
    mesh=_sc_mesh(),
    compiler_params=_sc_params,
    scratch_types=[
        pltpu.VMEM((EPW,), jnp.int32),
        pltpu.VMEM((EPW,), jnp.int32),
        pltpu.VMEM((2, SBC, K, HIDDEN), jnp.float32),
        pltpu.VMEM_SHARED((NP, HIDDEN), jnp.float32),
        pltpu.SemaphoreType.DMA((2,)),
        pltpu.SemaphoreType.DMA((2,)),
    ],
)


def _tc1a_body(x, w1, hq):
    h = jnp.dot(x[...], w1[...], preferred_element_type=jnp.float32)
    blocks = [h[i * NQ:(i + 1) * NQ, :] for i in range(7)]
    tail = jnp.concatenate(
        [h[7 * NQ:N_NODES, :],
         jnp.zeros((NP - N_NODES, HIDDEN), jnp.float32)], axis=0)
    hq[...] = jnp.concatenate(blocks + [tail], axis=1)


def _tc1b_body(hq, degq, hsq, dinvq):
    dinv = lax.rsqrt(degq[0] + degq[1] + 1.0)
    dinvq[...] = dinv
    hsq[...] = hq[...] * dinv


def _tc2_body(a1p, hsq, dinvq, b1t, gsq):
    a1 = (a1p[0] + a1p[1] + hsq[...]) * dinvq[...] + b1t[...]
    gsq[...] = jnp.maximum(a1, 0.0) * dinvq[...]


def _tc3_body(a2p, gsq, dinvq, w2b, b2t, out):
    a2 = (a2p[0] + a2p[1] + gsq[...]) * dinvq[...]
    out[...] = jnp.dot(a2, w2b[...], preferred_element_type=jnp.float32) \
        + b2t[...]


_tc1a = pl.pallas_call(
    _tc1a_body,
    out_shape=jax.ShapeDtypeStruct((NQ, 128), jnp.float32),
)

_tc1b = pl.pallas_call(
    _tc1b_body,
    out_shape=[jax.ShapeDtypeStruct((NQ, 128), jnp.float32),
               jax.ShapeDtypeStruct((NQ, 128), jnp.float32)],
)

_tc2 = pl.pallas_call(
    _tc2_body,
    out_shape=jax.ShapeDtypeStruct((NQ, 128), jnp.float32),
)

_tc3 = pl.pallas_call(
    _tc3_body,
    out_shape=jax.ShapeDtypeStruct((NQ, 8 * OUT_FEATS), jnp.float32),
)


def kernel(x, edge_index, W1, b1, W2, b2):
    def _perm(u):
        u = u.astype(jnp.int32)
        t = u >> 8
        d = ((t << 3) + (t << 2) + t) >> 6
        m = u - ((d << 10) + (d << 8))
        return (m << 3) + d

    src_p = _perm(edge_index[0])
    dst_p = _perm(edge_index[1])
    w2b = jnp.kron(jnp.eye(8, dtype=jnp.float32), W2)
    b1t = jnp.tile(b1, 8).reshape(1, 128)
    b2t = jnp.tile(b2, 8).reshape(1, 8 * OUT_FEATS)
    onesKH = jnp.ones((K, HIDDEN), jnp.float32)
    zerosRH = jnp.zeros((RPT, HIDDEN), jnp.float32)

    degp = _deg_call(dst_p, onesKH, zerosRH)
    hq = _tc1a(x, W1)
    hsq, dinvq = _tc1b(hq, degp.reshape(NC, NQ, 128))
    a1p = _agg_call(hsq.reshape(NP, HIDDEN), src_p, dst_p, zerosRH)
    gsq = _tc2(a1p.reshape(NC, NQ, 128), hsq, dinvq, b1t)
    a2p = _agg_call(gsq.reshape(NP, HIDDEN), src_p, dst_p, zerosRH)
    outq = _tc3(a2p.reshape(NC, NQ, 128), gsq, dinvq, w2b, b2t)
    o = outq.reshape(NQ, 8, OUT_FEATS).transpose(1, 0, 2)
    return o.reshape(NP, OUT_FEATS)[:N_NODES]


# speedup vs baseline: 1.1696x; 1.1224x over previous
import jax
import jax.numpy as jnp
from jax import lax
from jax.experimental import pallas as pl
from jax.experimental.pallas import tpu as pltpu
from jax.experimental.pallas import tpu_sc as plsc

N_NODES = 10000
N_EDGES = 320000
IN_FEATS = 128
HIDDEN = 16
OUT_FEATS = 2

NC, NS = 2, 16
NW = NC * NS
NP = 10240
NQ = NP // 8
RPT = NP // NS
EPW = N_EDGES // NW
K = 80
NCHUNK = EPW // K
SBC = 25
NSB = NCHUNK // SBC


def _sc_mesh():
    return plsc.VectorSubcoreMesh(
        core_axis_name="c", subcore_axis_name="s",
        num_cores=NC, num_subcores=NS)


def _deg_body(dst_hbm, ones_hbm, zeros_hbm, out_hbm, dst_v, ones_v, acc_sh,
              ssem):
    c = lax.axis_index("c")
    s = lax.axis_index("s")
    wid = s * NC + c
    pltpu.sync_copy(zeros_hbm, acc_sh.at[pl.ds(s * RPT, RPT)])
    pltpu.sync_copy(dst_hbm.at[pl.ds(wid * EPW, EPW)], dst_v)
    pltpu.sync_copy(ones_hbm, ones_v)
    plsc.subcore_barrier()

    def fire(j, carry):
        pltpu.async_copy(ones_v, acc_sh.at[dst_v.at[pl.ds(j * K, K)]],
                         ssem, add=True)
        return carry
    lax.fori_loop(0, NCHUNK, fire, 0)

    def drain(j, carry):
        pltpu.make_async_copy(ones_hbm, ones_v, ssem).wait()
        return carry
    lax.fori_loop(0, NCHUNK, drain, 0)

    plsc.subcore_barrier()
    pltpu.sync_copy(acc_sh.at[pl.ds(s * RPT, RPT)],
                    out_hbm.at[c].at[pl.ds(s * RPT, RPT)])


def _agg_body(rows_hbm, src_hbm, dst_hbm, zeros_hbm, out_hbm,
              src_v, dst_v, bufs, acc_sh, gsems, ssems):
    c = lax.axis_index("c")
    s = lax.axis_index("s")
    wid = s * NC + c
    pltpu.sync_copy(zeros_hbm, acc_sh.at[pl.ds(s * RPT, RPT)])
    pltpu.sync_copy(src_hbm.at[pl.ds(wid * EPW, EPW)], src_v)
    pltpu.sync_copy(dst_hbm.at[pl.ds(wid * EPW, EPW)], dst_v)
    plsc.subcore_barrier()

    def fire_gathers(sb, bi):
        base = sb * SBC
        def f(j, carry):
            idx = src_v.at[pl.ds((base + j) * K, K)]
            pltpu.async_copy(rows_hbm.at[idx], bufs.at[bi].at[j], gsems.at[bi])
            return carry
        lax.fori_loop(0, SBC, f, 0)

    def fire_scatters(sb, bi):
        base = sb * SBC
        def f(j, carry):
            idx = dst_v.at[pl.ds((base + j) * K, K)]
            pltpu.async_copy(bufs.at[bi].at[j], acc_sh.at[idx],
                             ssems.at[bi], add=True)
            return carry
        lax.fori_loop(0, SBC, f, 0)

    def drain(sem):
        def f(j, carry):
            pltpu.make_async_copy(rows_hbm.at[pl.ds(0, K)],
                                  bufs.at[0].at[0], sem).wait()
            return carry
        lax.fori_loop(0, SBC, f, 0)

    fire_gathers(0, 0)
    for sb in range(NSB):
        bi = sb % 2
        drain(gsems.at[bi])
        if sb >= 1:
            drain(ssems.at[(sb - 1) % 2])
        if sb + 1 < NSB:
            fire_gathers(sb + 1, (sb + 1) % 2)
        fire_scatters(sb, bi)
    drain(ssems.at[(NSB - 1) % 2])

    plsc.subcore_barrier()
    pltpu.sync_copy(acc_sh.at[pl.ds(s * RPT, RPT)],
                    out_hbm.at[c].at[pl.ds(s * RPT, RPT)])


_sc_params = pltpu.CompilerParams(use_tc_tiling_on_sc=False)

_deg_call = pl.kernel(
    _deg_body,
    out_type=jax.ShapeDtypeStruct((NC, NP, HIDDEN), jnp.float32),
    mesh=_sc_mesh(),
    compiler_params=_sc_params,
    scratch_types=[
        pltpu.VMEM((EPW,), jnp.int32),
        pltpu.VMEM((K, HIDDEN), jnp.float32),
        pltpu.VMEM_SHARED((NP, HIDDEN), jnp.float32),
        pltpu.SemaphoreType.DMA,
    ],
)

_agg_call = pl.kernel(
    _agg_body,
    out_type=jax.ShapeDtypeStruct((NC, NP, HIDDEN), jnp.float32),
    mesh=_sc_mesh(),
    compiler_params=_sc_params,
    scratch_types=[
        pltpu.VMEM((EPW,), jnp.int32),
        pltpu.VMEM((EPW,), jnp.int32),
        pltpu.VMEM((2, SBC, K, HIDDEN), jnp.float32),
        pltpu.VMEM_SHARED((NP, HIDDEN), jnp.float32),
        pltpu.SemaphoreType.DMA((2,)),
        pltpu.SemaphoreType.DMA((2,)),
    ],
)


def _perm_u(u):
    t = u >> 8
    d = ((t << 3) + (t << 2) + t) >> 6
    m = u - ((d << 10) + (d << 8))
    return (m << 3) + d


def _tc0_body(e_ref, src_o, dst_o):
    src_o[...] = _perm_u(e_ref[0, :])
    dst_o[...] = _perm_u(e_ref[1, :])


def _tc1a_body(x, w1, hq):
    h = jnp.dot(x[...], w1[...], preferred_element_type=jnp.float32)
    blocks = [h[i * NQ:(i + 1) * NQ, :] for i in range(7)]
    tail = jnp.concatenate(
        [h[7 * NQ:N_NODES, :],
         jnp.zeros((NP - N_NODES, HIDDEN), jnp.float32)], axis=0)
    hq[...] = jnp.concatenate(blocks + [tail], axis=1)


def _tc1b_body(hq, degq, hsq, dinvq):
    dinv = lax.rsqrt(degq[0] + degq[1] + 1.0)
    dinvq[...] = dinv
    hsq[...] = hq[...] * dinv


def _tc2_body(a1p, hsq, dinvq, b1t, gsq):
    a1 = (a1p[0] + a1p[1] + hsq[...]) * dinvq[...] + b1t[...]
    gsq[...] = jnp.maximum(a1, 0.0) * dinvq[...]


def _tc3_body(a2p, gsq, dinvq, w2b, b2t, out):
    a2 = (a2p[0] + a2p[1] + gsq[...]) * dinvq[...]
    out[...] = jnp.dot(a2, w2b[...], preferred_element_type=jnp.float32) \
        + b2t[...]


_tc0 = pl.pallas_call(
    _tc0_body,
    out_shape=[jax.ShapeDtypeStruct((N_EDGES,), jnp.int32),
               jax.ShapeDtypeStruct((N_EDGES,), jnp.int32)],
)

_tc1a = pl.pallas_call(
    _tc1a_body,
    out_shape=jax.ShapeDtypeStruct((NQ, 128), jnp.float32),
)

_tc1b = pl.pallas_call(
    _tc1b_body,
    out_shape=[jax.ShapeDtypeStruct((NQ, 128), jnp.float32),
               jax.ShapeDtypeStruct((NQ, 128), jnp.float32)],
)

_tc2 = pl.pallas_call(
    _tc2_body,
    out_shape=jax.ShapeDtypeStruct((NQ, 128), jnp.float32),
)

_tc3 = pl.pallas_call(
    _tc3_body,
    out_shape=jax.ShapeDtypeStruct((NQ, 8 * OUT_FEATS), jnp.float32),
)


def kernel(x, edge_index, W1, b1, W2, b2):
    src_p, dst_p = _tc0(edge_index.astype(jnp.int32))
    w2b = jnp.kron(jnp.eye(8, dtype=jnp.float32), W2)
    b1t = jnp.tile(b1, 8).reshape(1, 128)
    b2t = jnp.tile(b2, 8).reshape(1, 8 * OUT_FEATS)
    onesKH = jnp.ones((K, HIDDEN), jnp.float32)
    zerosRH = jnp.zeros((RPT, HIDDEN), jnp.float32)

    degp = _deg_call(dst_p, onesKH, zerosRH)
    hq = _tc1a(x, W1)
    hsq, dinvq = _tc1b(hq, degp.reshape(NC, NQ, 128))
    a1p = _agg_call(hsq.reshape(NP, HIDDEN), src_p, dst_p, zerosRH)
    gsq = _tc2(a1p.reshape(NC, NQ, 128), hsq, dinvq, b1t)
    a2p = _agg_call(gsq.reshape(NP, HIDDEN), src_p, dst_p, zerosRH)
    outq = _tc3(a2p.reshape(NC, NQ, 128), gsq, dinvq, w2b, b2t)
    o = outq.reshape(NQ, 8, OUT_FEATS).transpose(1, 0, 2)
    return o.reshape(NP, OUT_FEATS)[:N_NODES]


# speedup vs baseline: 1.1917x; 1.0188x over previous
import jax
import jax.numpy as jnp
from jax import lax
from jax.experimental import pallas as pl
from jax.experimental.pallas import tpu as pltpu
from jax.experimental.pallas import tpu_sc as plsc

N_NODES = 10000
N_EDGES = 320000
IN_FEATS = 128
HIDDEN = 16
OUT_FEATS = 2

NC, NS = 2, 16
NW = NC * NS
NP = 10240
NQ = NP // 8
RPT = NP // NS
EPW = N_EDGES // NW
K = 80
NCHUNK = EPW // K
SBC = 25
NSB = NCHUNK // SBC


def _sc_mesh():
    return plsc.VectorSubcoreMesh(
        core_axis_name="c", subcore_axis_name="s",
        num_cores=NC, num_subcores=NS)


def _deg_body(dst_hbm, ones_hbm, zeros_hbm, out_hbm, dst_v, ones_v, acc_sh,
              ssem):
    c = lax.axis_index("c")
    s = lax.axis_index("s")
    wid = s * NC + c
    pltpu.sync_copy(zeros_hbm, acc_sh.at[pl.ds(s * RPT, RPT)])
    pltpu.sync_copy(dst_hbm.at[pl.ds(wid * EPW, EPW)], dst_v)
    pltpu.sync_copy(ones_hbm, ones_v)
    plsc.subcore_barrier()

    def fire(j, carry):
        pltpu.async_copy(ones_v, acc_sh.at[dst_v.at[pl.ds(j * K, K)]],
                         ssem, add=True)
        return carry
    lax.fori_loop(0, NCHUNK, fire, 0)

    def drain(j, carry):
        pltpu.make_async_copy(ones_hbm, ones_v, ssem).wait()
        return carry
    lax.fori_loop(0, NCHUNK, drain, 0)

    plsc.subcore_barrier()
    pltpu.sync_copy(acc_sh.at[pl.ds(s * RPT, RPT)],
                    out_hbm.at[c].at[pl.ds(s * RPT, RPT)])


def _agg_body(rows_hbm, src_hbm, dst_hbm, zeros_hbm, out_hbm,
              src_v, dst_v, bufs, acc_sh, rows_sh, gsems, ssems):
    c = lax.axis_index("c")
    s = lax.axis_index("s")
    wid = s * NC + c
    pltpu.sync_copy(zeros_hbm, acc_sh.at[pl.ds(s * RPT, RPT)])
    pltpu.sync_copy(src_hbm.at[pl.ds(wid * EPW, EPW)], src_v)
    pltpu.sync_copy(dst_hbm.at[pl.ds(wid * EPW, EPW)], dst_v)
    pltpu.sync_copy(rows_hbm.at[pl.ds(s * RPT, RPT)],
                    rows_sh.at[pl.ds(s * RPT, RPT)])
    plsc.subcore_barrier()

    def fire_gathers(sb, bi):
        base = sb * SBC
        def f(j, carry):
            idx = src_v.at[pl.ds((base + j) * K, K)]
            pltpu.async_copy(rows_sh.at[idx], bufs.at[bi].at[j], gsems.at[bi])
            return carry
        lax.fori_loop(0, SBC, f, 0)

    def fire_scatters(sb, bi):
        base = sb * SBC
        def f(j, carry):
            idx = dst_v.at[pl.ds((base + j) * K, K)]
            pltpu.async_copy(bufs.at[bi].at[j], acc_sh.at[idx],
                             ssems.at[bi], add=True)
            return carry
        lax.fori_loop(0, SBC, f, 0)

    def drain(sem):
        def f(j, carry):
            pltpu.make_async_copy(rows_hbm.at[pl.ds(0, K)],
                                  bufs.at[0].at[0], sem).wait()
            return carry
        lax.fori_loop(0, SBC, f, 0)

    fire_gathers(0, 0)
    for sb in range(NSB):
        bi = sb % 2
        drain(gsems.at[bi])
        if sb >= 1:
            drain(ssems.at[(sb - 1) % 2])
        if sb + 1 < NSB:
            fire_gathers(sb + 1, (sb + 1) % 2)
        fire_scatters(sb, bi)
    drain(ssems.at[(NSB - 1) % 2])

    plsc.subcore_barrier()
    pltpu.sync_copy(acc_sh.at[pl.ds(s * RPT, RPT)],
                    out_hbm.at[c].at[pl.ds(s * RPT, RPT)])


_sc_params = pltpu.CompilerParams(use_tc_tiling_on_sc=False)

_deg_call = pl.kernel(
    _deg_body,
    out_type=jax.ShapeDtypeStruct((NC, NP, HIDDEN), jnp.float32),
    mesh=_sc_mesh(),
    compiler_params=_sc_params,
    scratch_types=[
        pltpu.VMEM((EPW,), jnp.int32),
        pltpu.VMEM((K, HIDDEN), jnp.float32),
        pltpu.VMEM_SHARED((NP, HIDDEN), jnp.float32),
        pltpu.SemaphoreType.DMA,
    ],
)

_agg_call = pl.kernel(
    _agg_body,
    out_type=jax.ShapeDtypeStruct((NC, NP, HIDDEN), jnp.float32),
    mesh=_sc_mesh(),
    compiler_params=_sc_params,
    scratch_types=[
        pltpu.VMEM((EPW,), jnp.int32),
        pltpu.VMEM((EPW,), jnp.int32),
        pltpu.VMEM((2, SBC, K, HIDDEN), jnp.float32),
        pltpu.VMEM_SHARED((NP, HIDDEN), jnp.float32),
        pltpu.VMEM_SHARED((NP, HIDDEN), jnp.float32),
        pltpu.SemaphoreType.DMA((2,)),
        pltpu.SemaphoreType.DMA((2,)),
    ],
)


def _perm_u(u):
    t = u >> 8
    d = ((t << 3) + (t << 2) + t) >> 6
    m = u - ((d << 10) + (d << 8))
    return (m << 3) + d


def _tc0_body(e_ref, src_o, dst_o):
    src_o[...] = _perm_u(e_ref[0, :])
    dst_o[...] = _perm_u(e_ref[1, :])


def _tc1a_body(x, w1, hq):
    h = jnp.dot(x[...], w1[...], preferred_element_type=jnp.float32)
    blocks = [h[i * NQ:(i + 1) * NQ, :] for i in range(7)]
    tail = jnp.concatenate(
        [h[7 * NQ:N_NODES, :],
         jnp.zeros((NP - N_NODES, HIDDEN), jnp.float32)], axis=0)
    hq[...] = jnp.concatenate(blocks + [tail], axis=1)


def _tc1b_body(hq, degq, hsq, dinvq):
    dinv = lax.rsqrt(degq[0] + degq[1] + 1.0)
    dinvq[...] = dinv
    hsq[...] = hq[...] * dinv


def _tc2_body(a1p, hsq, dinvq, b1t, gsq):
    a1 = (a1p[0] + a1p[1] + hsq[...]) * dinvq[...] + b1t[...]
    gsq[...] = jnp.maximum(a1, 0.0) * dinvq[...]


def _tc3_body(a2p, gsq, dinvq, w2b, b2t, out):
    a2 = (a2p[0] + a2p[1] + gsq[...]) * dinvq[...]
    out[...] = jnp.dot(a2, w2b[...], preferred_element_type=jnp.float32) \
        + b2t[...]


_tc0 = pl.pallas_call(
    _tc0_body,
    out_shape=[jax.ShapeDtypeStruct((N_EDGES,), jnp.int32),
               jax.ShapeDtypeStruct((N_EDGES,), jnp.int32)],
)

_tc1a = pl.pallas_call(
    _tc1a_body,
    out_shape=jax.ShapeDtypeStruct((NQ, 128), jnp.float32),
)

_tc1b = pl.pallas_call(
    _tc1b_body,
    out_shape=[jax.ShapeDtypeStruct((NQ, 128), jnp.float32),
               jax.ShapeDtypeStruct((NQ, 128), jnp.float32)],
)

_tc2 = pl.pallas_call(
    _tc2_body,
    out_shape=jax.ShapeDtypeStruct((NQ, 128), jnp.float32),
)

_tc3 = pl.pallas_call(
    _tc3_body,
    out_shape=jax.ShapeDtypeStruct((NQ, 8 * OUT_FEATS), jnp.float32),
)


def kernel(x, edge_index, W1, b1, W2, b2):
    src_p, dst_p = _tc0(edge_index.astype(jnp.int32))
    w2b = jnp.kron(jnp.eye(8, dtype=jnp.float32), W2)
    b1t = jnp.tile(b1, 8).reshape(1, 128)
    b2t = jnp.tile(b2, 8).reshape(1, 8 * OUT_FEATS)
    onesKH = jnp.ones((K, HIDDEN), jnp.float32)
    zerosRH = jnp.zeros((RPT, HIDDEN), jnp.float32)

    degp = _deg_call(dst_p, onesKH, zerosRH)
    hq = _tc1a(x, W1)
    hsq, dinvq = _tc1b(hq, degp.reshape(NC, NQ, 128))
    a1p = _agg_call(hsq.reshape(NP, HIDDEN), src_p, dst_p, zerosRH)
    gsq = _tc2(a1p.reshape(NC, NQ, 128), hsq, dinvq, b1t)
    a2p = _agg_call(gsq.reshape(NP, HIDDEN), src_p, dst_p, zerosRH)
    outq = _tc3(a2p.reshape(NC, NQ, 128), gsq, dinvq, w2b, b2t)
    o = outq.reshape(NQ, 8, OUT_FEATS).transpose(1, 0, 2)
    return o.reshape(NP, OUT_FEATS)[:N_NODES]


# speedup vs baseline: 1.3185x; 1.1064x over previous
import jax
import jax.numpy as jnp
from jax import lax
from jax.experimental import pallas as pl
from jax.experimental.pallas import tpu as pltpu
from jax.experimental.pallas import tpu_sc as plsc

N_NODES = 10000
N_EDGES = 320000
IN_FEATS = 128
HIDDEN = 16
OUT_FEATS = 2

NC, NS = 2, 16
NW = NC * NS
NP = 10240
NQ = NP // 8
RPT = NP // NS
EPW = N_EDGES // NW
K = 80
NCHUNK = EPW // K
DG = 16
DS = 8


def _sc_mesh():
    return plsc.VectorSubcoreMesh(
        core_axis_name="c", subcore_axis_name="s",
        num_cores=NC, num_subcores=NS)


def _deg_body(dst_hbm, ones_hbm, zeros_hbm, out_hbm, dst_v, ones_v, acc_sh,
              ssem):
    c = lax.axis_index("c")
    s = lax.axis_index("s")
    wid = s * NC + c
    pltpu.sync_copy(zeros_hbm, acc_sh.at[pl.ds(s * RPT, RPT)])
    pltpu.sync_copy(dst_hbm.at[pl.ds(wid * EPW, EPW)], dst_v)
    pltpu.sync_copy(ones_hbm, ones_v)
    plsc.subcore_barrier()

    def fire(j, carry):
        pltpu.async_copy(ones_v, acc_sh.at[dst_v.at[pl.ds(j * K, K)]],
                         ssem, add=True)
        return carry
    lax.fori_loop(0, NCHUNK, fire, 0)

    def drain(j, carry):
        pltpu.make_async_copy(ones_hbm, ones_v, ssem).wait()
        return carry
    lax.fori_loop(0, NCHUNK, drain, 0)

    plsc.subcore_barrier()
    pltpu.sync_copy(acc_sh.at[pl.ds(s * RPT, RPT)],
                    out_hbm.at[c].at[pl.ds(s * RPT, RPT)])


def _agg_body(rows_hbm, src_hbm, dst_hbm, zeros_hbm, out_hbm,
              src_v, dst_v, bufs, acc_sh, rows_sh, gsem, ssem):
    c = lax.axis_index("c")
    s = lax.axis_index("s")
    wid = s * NC + c
    pltpu.sync_copy(zeros_hbm, acc_sh.at[pl.ds(s * RPT, RPT)])
    pltpu.sync_copy(src_hbm.at[pl.ds(wid * EPW, EPW)], src_v)
    pltpu.sync_copy(dst_hbm.at[pl.ds(wid * EPW, EPW)], dst_v)
    pltpu.sync_copy(rows_hbm.at[pl.ds(s * RPT, RPT)],
                    rows_sh.at[pl.ds(s * RPT, RPT)])
    plsc.subcore_barrier()

    def fire_gather(j):
        idx = src_v.at[pl.ds(j * K, K)]
        pltpu.async_copy(rows_sh.at[idx], bufs.at[j % DG], gsem)

    def fire_scatter(j):
        idx = dst_v.at[pl.ds(j * K, K)]
        pltpu.async_copy(bufs.at[j % DG], acc_sh.at[idx], ssem, add=True)

    def wait1(sem):
        pltpu.make_async_copy(rows_hbm.at[pl.ds(0, K)],
                              bufs.at[0], sem).wait()

    def prime(j, carry):
        fire_gather(j)
        return carry
    lax.fori_loop(0, DG, prime, 0)

    def head(j, carry):
        wait1(gsem)
        fire_scatter(j)
        return carry
    lax.fori_loop(0, DS, head, 0)

    def steady(j, carry):
        wait1(gsem)
        fire_scatter(j)
        wait1(ssem)
        fire_gather(j + DG - DS)
        return carry
    lax.fori_loop(DS, NCHUNK - DG + DS, steady, 0)

    def tail(j, carry):
        wait1(gsem)
        fire_scatter(j)
        wait1(ssem)
        return carry
    lax.fori_loop(NCHUNK - DG + DS, NCHUNK, tail, 0)

    def flush(j, carry):
        wait1(ssem)
        return carry
    lax.fori_loop(0, DS, flush, 0)

    plsc.subcore_barrier()
    pltpu.sync_copy(acc_sh.at[pl.ds(s * RPT, RPT)],
                    out_hbm.at[c].at[pl.ds(s * RPT, RPT)])


_sc_params = pltpu.CompilerParams(use_tc_tiling_on_sc=False)

_deg_call = pl.kernel(
    _deg_body,
    out_type=jax.ShapeDtypeStruct((NC, NP, HIDDEN), jnp.float32),
    mesh=_sc_mesh(),
    compiler_params=_sc_params,
    scratch_types=[
        pltpu.VMEM((EPW,), jnp.int32),
        pltpu.VMEM((K, HIDDEN), jnp.float32),
        pltpu.VMEM_SHARED((NP, HIDDEN), jnp.float32),
        pltpu.SemaphoreType.DMA,
    ],
)

_agg_call = pl.kernel(
    _agg_body,
    out_type=jax.ShapeDtypeStruct((NC, NP, HIDDEN), jnp.float32),
    mesh=_sc_mesh(),
    compiler_params=_sc_params,
    scratch_types=[
        pltpu.VMEM((EPW,), jnp.int32),
        pltpu.VMEM((EPW,), jnp.int32),
        pltpu.VMEM((DG, K, HIDDEN), jnp.float32),
        pltpu.VMEM_SHARED((NP, HIDDEN), jnp.float32),
        pltpu.VMEM_SHARED((NP, HIDDEN), jnp.float32),
        pltpu.SemaphoreType.DMA,
        pltpu.SemaphoreType.DMA,
    ],
)


def _perm_u(u):
    t = u >> 8
    d = ((t << 3) + (t << 2) + t) >> 6
    m = u - ((d << 10) + (d << 8))
    return (m << 3) + d


def _tc0_body(e_ref, src_o, dst_o):
    src_o[...] = _perm_u(e_ref[0, :])
    dst_o[...] = _perm_u(e_ref[1, :])


def _tc1a_body(x, w1, hq):
    h = jnp.dot(x[...], w1[...], preferred_element_type=jnp.float32)
    blocks = [h[i * NQ:(i + 1) * NQ, :] for i in range(7)]
    tail = jnp.concatenate(
        [h[7 * NQ:N_NODES, :],
         jnp.zeros((NP - N_NODES, HIDDEN), jnp.float32)], axis=0)
    hq[...] = jnp.concatenate(blocks + [tail], axis=1)


def _tc1b_body(hq, degq, hsq, dinvq):
    dinv = lax.rsqrt(degq[0] + degq[1] + 1.0)
    dinvq[...] = dinv
    hsq[...] = hq[...] * dinv


def _tc2_body(a1p, hsq, dinvq, b1t, gsq):
    a1 = (a1p[0] + a1p[1] + hsq[...]) * dinvq[...] + b1t[...]
    gsq[...] = jnp.maximum(a1, 0.0) * dinvq[...]


def _tc3_body(a2p, gsq, dinvq, w2b, b2t, out):
    a2 = (a2p[0] + a2p[1] + gsq[...]) * dinvq[...]
    out[...] = jnp.dot(a2, w2b[...], preferred_element_type=jnp.float32) \
        + b2t[...]


_tc0 = pl.pallas_call(
    _tc0_body,
    out_shape=[jax.ShapeDtypeStruct((N_EDGES,), jnp.int32),
               jax.ShapeDtypeStruct((N_EDGES,), jnp.int32)],
)

_tc1a = pl.pallas_call(
    _tc1a_body,
    out_shape=jax.ShapeDtypeStruct((NQ, 128), jnp.float32),
)

_tc1b = pl.pallas_call(
    _tc1b_body,
    out_shape=[jax.ShapeDtypeStruct((NQ, 128), jnp.float32),
               jax.ShapeDtypeStruct((NQ, 128), jnp.float32)],
)

_tc2 = pl.pallas_call(
    _tc2_body,
    out_shape=jax.ShapeDtypeStruct((NQ, 128), jnp.float32),
)

_tc3 = pl.pallas_call(
    _tc3_body,
    out_shape=jax.ShapeDtypeStruct((NQ, 8 * OUT_FEATS), jnp.float32),
)


def kernel(x, edge_index, W1, b1, W2, b2):
    src_p, dst_p = _tc0(edge_index.astype(jnp.int32))
    w2b = jnp.kron(jnp.eye(8, dtype=jnp.float32), W2)
    b1t = jnp.tile(b1, 8).reshape(1, 128)
    b2t = jnp.tile(b2, 8).reshape(1, 8 * OUT_FEATS)
    onesKH = jnp.ones((K, HIDDEN), jnp.float32)
    zerosRH = jnp.zeros((RPT, HIDDEN), jnp.float32)

    degp = _deg_call(dst_p, onesKH, zerosRH)
    hq = _tc1a(x, W1)
    hsq, dinvq = _tc1b(hq, degp.reshape(NC, NQ, 128))
    a1p = _agg_call(hsq.reshape(NP, HIDDEN), src_p, dst_p, zerosRH)
    gsq = _tc2(a1p.reshape(NC, NQ, 128), hsq, dinvq, b1t)
    a2p = _agg_call(gsq.reshape(NP, HIDDEN), src_p, dst_p, zerosRH)
    outq = _tc3(a2p.reshape(NC, NQ, 128), gsq, dinvq, w2b, b2t)
    o = outq.reshape(NQ, 8, OUT_FEATS).transpose(1, 0, 2)
    return o.reshape(NP, OUT_FEATS)[:N_NODES]
